# baseline (speedup 1.0000x reference)
import functools

import jax
import jax.numpy as jnp
from jax import lax
from jax.experimental import pallas as pl
from jax.experimental.pallas import tpu as pltpu
from jax.experimental.pallas import tpu_sc as plsc

N_S, N_C, E, D = 8000, 2000, 80000, 128

NC, NS, CHUNK = 2, 16, 256
E_PAD = 81920
NCHW = E_PAD // (NS * CHUNK)
EPW = E_PAD // NS
TS_ROWS, TC_ROWS = 8064, 2048
ZS, ZC = TS_ROWS // NS, TC_ROWS // NS

PER_S, LAST_S = 504, N_S - 15 * 504
PER_C, LAST_C = 128, N_C - 15 * 128


def _mo8(x):
    return pl.multiple_of(x, 8)

_mesh = plsc.VectorSubcoreMesh(core_axis_name="c", subcore_axis_name="s",
                               num_cores=NC, num_subcores=NS)


def _f32(shape):
    return jax.ShapeDtypeStruct(shape, jnp.float32)


@functools.partial(
    pl.kernel,
    out_type=[_f32((N_S, D)), _f32((N_C, D)), _f32((N_S, D)), _f32((N_C, D))],
    mesh=_mesh,
    scratch_types=[
        pltpu.VMEM_SHARED((TS_ROWS, D), jnp.float32),
        pltpu.VMEM_SHARED((TC_ROWS, D), jnp.float32),
        pltpu.VMEM((EPW,), jnp.int32),
        pltpu.VMEM((CHUNK, D), jnp.float32),
    ],
)
def _hist2(e0, e1, e2, e3, ones_hbm, zeros_hbm, h0, h1, h2, h3,
           ts_tab, tc_tab, idx_v, ones_v):
    sid = lax.axis_index("s")
    cid = lax.axis_index("c")

    pltpu.sync_copy(ones_hbm, ones_v)
    pltpu.sync_copy(zeros_hbm.at[pl.ds(0, ZS)],
                    ts_tab.at[pl.ds(_mo8(sid * ZS), ZS)])
    pltpu.sync_copy(zeros_hbm.at[pl.ds(0, ZC)],
                    tc_tab.at[pl.ds(_mo8(sid * ZC), ZC)])
    plsc.subcore_barrier()

    def hist(e1d, table):
        pltpu.sync_copy(e1d.at[pl.ds(pl.multiple_of(sid * EPW, 256), EPW)],
                        idx_v)

        @pl.loop(0, NCHW)
        def _(j):
            sl = pl.ds(pl.multiple_of(j * CHUNK, 256), CHUNK)
            pltpu.sync_copy(ones_v, table.at[idx_v.at[sl]], add=True)

    @pl.when(cid == 0)
    def _():
        hist(e0, ts_tab)
        hist(e1, tc_tab)

    @pl.when(cid == 1)
    def _():
        hist(e2, ts_tab)
        hist(e3, tc_tab)

    plsc.subcore_barrier()

    def rd(table, out, per, last):
        @pl.when(sid < NS - 1)
        def _():
            off = _mo8(sid * per)
            pltpu.sync_copy(table.at[pl.ds(off, per)], out.at[pl.ds(off, per)])

        @pl.when(sid == NS - 1)
        def _():
            off = (NS - 1) * per
            pltpu.sync_copy(table.at[pl.ds(off, last)],
                            out.at[pl.ds(off, last)])

    @pl.when(cid == 0)
    def _():
        rd(ts_tab, h0, PER_S, LAST_S)
        rd(tc_tab, h1, PER_C, LAST_C)

    @pl.when(cid == 1)
    def _():
        rd(ts_tab, h2, PER_S, LAST_S)
        rd(tc_tab, h3, PER_C, LAST_C)


CH_A = 64
NCH_A = EPW // CH_A
NBUF = 4


@functools.partial(
    pl.kernel,
    out_type=[_f32((N_S, D)), _f32((N_C, D)),
              _f32((N_S, D)), _f32((N_C, D))],
    mesh=_mesh,
    scratch_types=[
        pltpu.VMEM_SHARED((TS_ROWS, D), jnp.float32),
        pltpu.VMEM_SHARED((TC_ROWS, D), jnp.float32),
        pltpu.VMEM((EPW,), jnp.int32),
        pltpu.VMEM((EPW,), jnp.int32),
        pltpu.VMEM((CH_A, D), jnp.float32),
        pltpu.VMEM((CH_A, D), jnp.float32),
        pltpu.VMEM((CH_A, D), jnp.float32),
        pltpu.VMEM((CH_A, D), jnp.float32),
        pltpu.SemaphoreType.DMA,
        pltpu.SemaphoreType.DMA,
        pltpu.SemaphoreType.DMA,
        pltpu.SemaphoreType.DMA,
    ],
)
def _aggregate(y_ss, y_sc, y_cs, y_cc,
               ss_s, ss_d, sc_s, sc_d, cs_s, cs_d, cc_s, cc_d,
               zeros_hbm,
               agg_ss, agg_sc, agg_cs, agg_cc,
               tab, stage, sidx, didx, rows0, rows1, rows2, rows3,
               sem0, sem1, sem2, sem3):
    sid = lax.axis_index("s")
    cid = lax.axis_index("c")

    def stripes(per, last, fn):
        @pl.when(sid < NS - 1)
        def _():
            fn(_mo8(sid * per), per)

        @pl.when(sid == NS - 1)
        def _():
            fn((NS - 1) * per, last)

    def rd(out, per, last):
        def f(off, sz):
            pltpu.sync_copy(tab.at[pl.ds(off, sz)], out.at[pl.ds(off, sz)])
        stripes(per, last, f)

    def load_stage(y_hbm):
        def f(off, sz):
            pltpu.sync_copy(y_hbm.at[pl.ds(off, sz)],
                            stage.at[pl.ds(off, sz)])
        stripes(PER_C, LAST_C, f)

    def do_rel(src, s1d, d1d):
        base = pl.ds(pl.multiple_of(sid * EPW, 256), EPW)
        pltpu.sync_copy(s1d.at[base], sidx)
        pltpu.sync_copy(d1d.at[base], didx)

        def g(j):
            return pl.ds(pl.multiple_of(j * CH_A, CH_A), CH_A)

        pltpu.make_async_copy(src.at[sidx.at[g(0)]], rows0, sem0).start()

        @pl.loop(0, NCH_A, step=2)
        def _(j):
            pltpu.make_async_copy(src.at[sidx.at[g(j)]], rows0, sem0).wait()
            pltpu.make_async_copy(src.at[sidx.at[g(j + 1)]], rows1,
                                  sem1).start()
            pltpu.sync_copy(rows0, tab.at[didx.at[g(j)]], add=True)
            pltpu.make_async_copy(src.at[sidx.at[g(j + 1)]], rows1,
                                  sem1).wait()

            @pl.when(j + 2 < NCH_A)
            def _():
                pltpu.make_async_copy(src.at[sidx.at[g(j + 2)]], rows0,
                                      sem0).start()

            pltpu.sync_copy(rows1, tab.at[didx.at[g(j + 1)]], add=True)

    pltpu.sync_copy(zeros_hbm.at[pl.ds(0, ZS)],
                    tab.at[pl.ds(_mo8(sid * ZS), ZS)])

    @pl.when(cid == 1)
    def _():
        load_stage(y_cs)

    plsc.subcore_barrier()

    @pl.when(cid == 0)
    def _():
        do_rel(y_ss, ss_s, ss_d)

    @pl.when(cid == 1)
    def _():
        do_rel(stage, cs_s, cs_d)

    plsc.subcore_barrier()

    @pl.when(cid == 0)
    def _():
        rd(agg_ss, PER_S, LAST_S)
        load_stage(y_cc)

    @pl.when(cid == 1)
    def _():
        rd(agg_cs, PER_S, LAST_S)

    plsc.subcore_barrier()

    pltpu.sync_copy(zeros_hbm.at[pl.ds(0, ZC)],
                    tab.at[pl.ds(_mo8(sid * ZC), ZC)])
    plsc.subcore_barrier()

    @pl.when(cid == 0)
    def _():
        do_rel(stage, cc_s, cc_d)

    @pl.when(cid == 1)
    def _():
        do_rel(y_sc, sc_s, sc_d)

    plsc.subcore_barrier()

    @pl.when(cid == 0)
    def _():
        rd(agg_cc, PER_C, LAST_C)

    @pl.when(cid == 1)
    def _():
        rd(agg_sc, PER_C, LAST_C)


_BLK = 1000


def _norm_col(deg_blk):
    return lax.rsqrt(jnp.maximum(deg_blk[:, :1], 1.0))


def _prep_body(feat_ref, dega_ref, degb_ref, ya_ref, yb_ref):
    f = feat_ref[...]
    ya_ref[...] = f * _norm_col(dega_ref[...])
    yb_ref[...] = f * _norm_col(degb_ref[...])


def _prep(feat, dega, degb):
    n = feat.shape[0]
    bs_x = pl.BlockSpec((_BLK, D), lambda i: (i, 0))
    bs_deg = pl.BlockSpec((_BLK, D), lambda i: (i, 0))
    return pl.pallas_call(
        _prep_body,
        grid=(n // _BLK,),
        in_specs=[bs_x, bs_deg, bs_deg],
        out_specs=[bs_x, bs_x],
        out_shape=[_f32((n, D)), _f32((n, D))],
    )(feat, dega, degb)


def _dense1_body(aa_ref, ab_ref, dia_ref, dib_ref, wa_ref, wb_ref,
                 ba_ref, bb_ref, doa_ref, dob_ref, ya_ref, yb_ref):
    a = aa_ref[...] * _norm_col(dia_ref[...])
    b = ab_ref[...] * _norm_col(dib_ref[...])
    h = (jnp.dot(a, wa_ref[...], preferred_element_type=jnp.float32)
         + jnp.dot(b, wb_ref[...], preferred_element_type=jnp.float32)
         + ba_ref[...] + bb_ref[...])
    h = jnp.maximum(h, 0.0)
    ya_ref[...] = h * _norm_col(doa_ref[...])
    yb_ref[...] = h * _norm_col(dob_ref[...])


def _dense1(agg_a, agg_b, deg_in_a, deg_in_b, wa, wb, ba, bb,
            deg_out_a, deg_out_b):
    n = agg_a.shape[0]
    bs_x = pl.BlockSpec((_BLK, D), lambda i: (i, 0))
    bs_deg = pl.BlockSpec((_BLK, D), lambda i: (i, 0))
    bs_w = pl.BlockSpec((D, D), lambda i: (0, 0))
    bs_b = pl.BlockSpec((1, D), lambda i: (0, 0))
    return pl.pallas_call(
        _dense1_body,
        grid=(n // _BLK,),
        in_specs=[bs_x, bs_x, bs_deg, bs_deg, bs_w, bs_w, bs_b, bs_b,
                  bs_deg, bs_deg],
        out_specs=[bs_x, bs_x],
        out_shape=[_f32((n, D)), _f32((n, D))],
    )(agg_a, agg_b, deg_in_a, deg_in_b, wa, wb, ba, bb,
      deg_out_a, deg_out_b)


def _final_s_body(aa_ref, ab_ref, dia_ref, dib_ref, wa_ref, wb_ref,
                  ba_ref, bb_ref, out_ref):
    i = pl.program_id(0)
    a = aa_ref[...] * _norm_col(dia_ref[...])
    b = ab_ref[...] * _norm_col(dib_ref[...])
    h = (jnp.dot(a, wa_ref[...], preferred_element_type=jnp.float32)
         + jnp.dot(b, wb_ref[...], preferred_element_type=jnp.float32)
         + ba_ref[...] + bb_ref[...])
    m = jnp.max(h, axis=0, keepdims=True)

    @pl.when(i == 0)
    def _():
        out_ref[...] = m

    @pl.when(i > 0)
    def _():
        out_ref[...] = jnp.maximum(out_ref[...], m)


def _final_s(agg_a, agg_b, deg_in_a, deg_in_b, wa, wb, ba, bb):
    bs_x = pl.BlockSpec((_BLK, D), lambda i: (i, 0))
    bs_deg = pl.BlockSpec((_BLK, D), lambda i: (i, 0))
    bs_w = pl.BlockSpec((D, D), lambda i: (0, 0))
    bs_b = pl.BlockSpec((1, D), lambda i: (0, 0))
    return pl.pallas_call(
        _final_s_body,
        grid=(N_S // _BLK,),
        in_specs=[bs_x, bs_x, bs_deg, bs_deg, bs_w, bs_w, bs_b, bs_b],
        out_specs=pl.BlockSpec((1, D), lambda i: (0, 0)),
        out_shape=_f32((1, D)),
    )(agg_a, agg_b, deg_in_a, deg_in_b, wa, wb, ba, bb)


def _final_c_body(aa_ref, ab_ref, dia_ref, dib_ref, wa_ref, wb_ref,
                  ba_ref, bb_ref, out_ref):
    a = aa_ref[...] * _norm_col(dia_ref[...])
    b = ab_ref[...] * _norm_col(dib_ref[...])
    out_ref[...] = (jnp.dot(a, wa_ref[...], preferred_element_type=jnp.float32)
                    + jnp.dot(b, wb_ref[...], preferred_element_type=jnp.float32)
                    + ba_ref[...] + bb_ref[...])


def _final_c(agg_a, agg_b, deg_in_a, deg_in_b, wa, wb, ba, bb):
    bs_x = pl.BlockSpec((_BLK, D), lambda i: (i, 0))
    bs_deg = pl.BlockSpec((_BLK, D), lambda i: (i, 0))
    bs_w = pl.BlockSpec((D, D), lambda i: (0, 0))
    bs_b = pl.BlockSpec((1, D), lambda i: (0, 0))
    return pl.pallas_call(
        _final_c_body,
        grid=(N_C // _BLK,),
        in_specs=[bs_x, bs_x, bs_deg, bs_deg, bs_w, bs_w, bs_b, bs_b],
        out_specs=bs_x,
        out_shape=_f32((N_C, D)),
    )(agg_a, agg_b, deg_in_a, deg_in_b, wa, wb, ba, bb)


def kernel(feat_sentence, feat_context, ss_src, ss_dst, sc_src, sc_dst,
           cs_src, cs_dst, cc_src, cc_dst,
           W1_ss, b1_ss, W1_sc, b1_sc, W1_cs, b1_cs, W1_cc, b1_cc,
           W2_ss, b2_ss, W2_sc, b2_sc, W2_cs, b2_cs, W2_cc, b2_cc):
    def pad2d(idx, fill):
        idx = idx.astype(jnp.int32)
        pad = jnp.full((E_PAD - E,), fill, jnp.int32)
        return jnp.concatenate([idx, pad])


    ss_sd, ss_d2 = pad2d(ss_src, N_S), pad2d(ss_dst, N_S)
    sc_sd, sc_d2 = pad2d(sc_src, N_S), pad2d(sc_dst, N_C)
    cs_sd, cs_d2 = pad2d(cs_src, N_C), pad2d(cs_dst, N_S)
    cc_sd, cc_d2 = pad2d(cc_src, N_C), pad2d(cc_dst, N_C)
    ss_s2, sc_s2 = pad2d(ss_src, 0), pad2d(sc_src, 0)
    cs_s2, cc_s2 = pad2d(cs_src, 0), pad2d(cc_src, 0)

    ones128 = jnp.ones((CHUNK, D), jnp.float32)
    zeros128 = jnp.zeros((ZS, D), jnp.float32)

    d_ss_dst, d_sc_dst, d_cs_dst, d_cc_dst = _hist2(
        ss_d2, sc_d2, cs_d2, cc_d2, ones128, zeros128)
    d_ss_src, d_cs_src, d_sc_src, d_cc_src = _hist2(
        ss_sd, cs_sd, sc_sd, cc_sd, ones128, zeros128)

    y_ss, y_sc = _prep(feat_sentence, d_ss_src, d_sc_src)
    y_cs, y_cc = _prep(feat_context, d_cs_src, d_cc_src)

    agg_ss, agg_sc, agg_cs, agg_cc = _aggregate(
        y_ss, y_sc, y_cs, y_cc,
        ss_s2, ss_d2, sc_s2, sc_d2, cs_s2, cs_d2, cc_s2, cc_d2, zeros128)

    b1_ss_r, b1_sc_r = b1_ss.reshape(1, D), b1_sc.reshape(1, D)
    b1_cs_r, b1_cc_r = b1_cs.reshape(1, D), b1_cc.reshape(1, D)
    b2_ss_r, b2_sc_r = b2_ss.reshape(1, D), b2_sc.reshape(1, D)
    b2_cs_r, b2_cc_r = b2_cs.reshape(1, D), b2_cc.reshape(1, D)

    y2_ss, y2_sc = _dense1(agg_ss, agg_cs, d_ss_dst, d_cs_dst,
                           W1_ss, W1_cs, b1_ss_r, b1_cs_r,
                           d_ss_src, d_sc_src)
    y2_cs, y2_cc = _dense1(agg_sc, agg_cc, d_sc_dst, d_cc_dst,
                           W1_sc, W1_cc, b1_sc_r, b1_cc_r,
                           d_cs_src, d_cc_src)

    agg2_ss, agg2_sc, agg2_cs, agg2_cc = _aggregate(
        y2_ss, y2_sc, y2_cs, y2_cc,
        ss_s2, ss_d2, sc_s2, sc_d2, cs_s2, cs_d2, cc_s2, cc_d2, zeros128)

    doc = _final_s(agg2_ss, agg2_cs, d_ss_dst, d_cs_dst,
                   W2_ss, W2_cs, b2_ss_r, b2_cs_r)
    h_c = _final_c(agg2_sc, agg2_cc, d_sc_dst, d_cc_dst,
                   W2_sc, W2_cc, b2_sc_r, b2_cc_r)
    return (doc, h_c)


# speedup vs baseline: 1.0560x; 1.0560x over previous
import functools

import jax
import jax.numpy as jnp
from jax import lax
from jax.experimental import pallas as pl
from jax.experimental.pallas import tpu as pltpu
from jax.experimental.pallas import tpu_sc as plsc

N_S, N_C, E, D = 8000, 2000, 80000, 128

NC, NS, CHUNK = 2, 16, 256
E_PAD = 81920
NCHW = E_PAD // (NS * CHUNK)
EPW = E_PAD // NS
TS_ROWS, TC_ROWS = 8064, 2048
ZS, ZC = TS_ROWS // NS, TC_ROWS // NS

PER_S, LAST_S = 504, N_S - 15 * 504
PER_C, LAST_C = 128, N_C - 15 * 128


def _mo8(x):
    return pl.multiple_of(x, 8)

_mesh = plsc.VectorSubcoreMesh(core_axis_name="c", subcore_axis_name="s",
                               num_cores=NC, num_subcores=NS)


def _f32(shape):
    return jax.ShapeDtypeStruct(shape, jnp.float32)


@functools.partial(
    pl.kernel,
    out_type=[_f32((N_S, D)), _f32((N_C, D)), _f32((N_S, D)), _f32((N_C, D))],
    mesh=_mesh,
    scratch_types=[
        pltpu.VMEM_SHARED((TS_ROWS, D), jnp.float32),
        pltpu.VMEM_SHARED((TC_ROWS, D), jnp.float32),
        pltpu.VMEM((EPW,), jnp.int32),
        pltpu.VMEM((CHUNK, D), jnp.float32),
    ],
)
def _hist2(e0, e1, e2, e3, ones_hbm, zeros_hbm, h0, h1, h2, h3,
           ts_tab, tc_tab, idx_v, ones_v):
    sid = lax.axis_index("s")
    cid = lax.axis_index("c")

    pltpu.sync_copy(ones_hbm, ones_v)
    pltpu.sync_copy(zeros_hbm.at[pl.ds(0, ZS)],
                    ts_tab.at[pl.ds(_mo8(sid * ZS), ZS)])
    pltpu.sync_copy(zeros_hbm.at[pl.ds(0, ZC)],
                    tc_tab.at[pl.ds(_mo8(sid * ZC), ZC)])
    plsc.subcore_barrier()

    def hist(e1d, table):
        pltpu.sync_copy(e1d.at[pl.ds(pl.multiple_of(sid * EPW, 256), EPW)],
                        idx_v)

        @pl.loop(0, NCHW)
        def _(j):
            sl = pl.ds(pl.multiple_of(j * CHUNK, 256), CHUNK)
            pltpu.sync_copy(ones_v, table.at[idx_v.at[sl]], add=True)

    @pl.when(cid == 0)
    def _():
        hist(e0, ts_tab)
        hist(e1, tc_tab)

    @pl.when(cid == 1)
    def _():
        hist(e2, ts_tab)
        hist(e3, tc_tab)

    plsc.subcore_barrier()

    def rd(table, out, per, last):
        @pl.when(sid < NS - 1)
        def _():
            off = _mo8(sid * per)
            pltpu.sync_copy(table.at[pl.ds(off, per)], out.at[pl.ds(off, per)])

        @pl.when(sid == NS - 1)
        def _():
            off = (NS - 1) * per
            pltpu.sync_copy(table.at[pl.ds(off, last)],
                            out.at[pl.ds(off, last)])

    @pl.when(cid == 0)
    def _():
        rd(ts_tab, h0, PER_S, LAST_S)
        rd(tc_tab, h1, PER_C, LAST_C)

    @pl.when(cid == 1)
    def _():
        rd(ts_tab, h2, PER_S, LAST_S)
        rd(tc_tab, h3, PER_C, LAST_C)


CH_A = 128
NCH_A = EPW // CH_A


@functools.partial(
    pl.kernel,
    out_type=[_f32((N_S, D)), _f32((N_C, D)),
              _f32((N_S, D)), _f32((N_C, D))],
    mesh=_mesh,
    scratch_types=[
        pltpu.VMEM_SHARED((TS_ROWS, D), jnp.float32),
        pltpu.VMEM_SHARED((TC_ROWS, D), jnp.float32),
        pltpu.VMEM((EPW,), jnp.int32),
        pltpu.VMEM((EPW,), jnp.int32),
        pltpu.VMEM((CH_A, D), jnp.float32),
        pltpu.VMEM((CH_A, D), jnp.float32),
        pltpu.SemaphoreType.DMA,
        pltpu.SemaphoreType.DMA,
    ],
)
def _aggregate(y_ss, y_sc, y_cs, y_cc,
               ss_s, ss_d, sc_s, sc_d, cs_s, cs_d, cc_s, cc_d,
               zeros_hbm,
               agg_ss, agg_sc, agg_cs, agg_cc,
               tab, stage, sidx, didx, rows0, rows1,
               sem0, sem1):
    sid = lax.axis_index("s")
    cid = lax.axis_index("c")

    def stripes(per, last, fn):
        @pl.when(sid < NS - 1)
        def _():
            fn(_mo8(sid * per), per)

        @pl.when(sid == NS - 1)
        def _():
            fn((NS - 1) * per, last)

    def rd(out, per, last):
        def f(off, sz):
            pltpu.sync_copy(tab.at[pl.ds(off, sz)], out.at[pl.ds(off, sz)])
        stripes(per, last, f)

    def load_stage(y_hbm, per, last):
        def f(off, sz):
            pltpu.sync_copy(y_hbm.at[pl.ds(off, sz)],
                            stage.at[pl.ds(off, sz)])
        stripes(per, last, f)

    def do_rel(src, s1d, d1d):
        base = pl.ds(pl.multiple_of(sid * EPW, 256), EPW)
        pltpu.sync_copy(s1d.at[base], sidx)
        pltpu.sync_copy(d1d.at[base], didx)

        def g(j):
            return pl.ds(pl.multiple_of(j * CH_A, CH_A), CH_A)

        pltpu.make_async_copy(src.at[sidx.at[g(0)]], rows0, sem0).start()

        @pl.loop(0, NCH_A, step=2)
        def _(j):
            pltpu.make_async_copy(src.at[sidx.at[g(j)]], rows0, sem0).wait()
            pltpu.make_async_copy(src.at[sidx.at[g(j + 1)]], rows1,
                                  sem1).start()
            pltpu.sync_copy(rows0, tab.at[didx.at[g(j)]], add=True)
            pltpu.make_async_copy(src.at[sidx.at[g(j + 1)]], rows1,
                                  sem1).wait()

            @pl.when(j + 2 < NCH_A)
            def _():
                pltpu.make_async_copy(src.at[sidx.at[g(j + 2)]], rows0,
                                      sem0).start()

            pltpu.sync_copy(rows1, tab.at[didx.at[g(j + 1)]], add=True)

    pltpu.sync_copy(zeros_hbm.at[pl.ds(0, ZS)],
                    tab.at[pl.ds(_mo8(sid * ZS), ZS)])

    @pl.when(cid == 1)
    def _():
        load_stage(y_cs, PER_C, LAST_C)

    plsc.subcore_barrier()

    @pl.when(cid == 0)
    def _():
        do_rel(y_ss, ss_s, ss_d)

    @pl.when(cid == 1)
    def _():
        do_rel(stage, cs_s, cs_d)

    plsc.subcore_barrier()

    @pl.when(cid == 0)
    def _():
        rd(agg_ss, PER_S, LAST_S)
        load_stage(y_cc, PER_C, LAST_C)

    @pl.when(cid == 1)
    def _():
        rd(agg_cs, PER_S, LAST_S)

    plsc.subcore_barrier()

    pltpu.sync_copy(zeros_hbm.at[pl.ds(0, ZC)],
                    tab.at[pl.ds(_mo8(sid * ZC), ZC)])
    plsc.subcore_barrier()

    @pl.when(cid == 0)
    def _():
        do_rel(stage, cc_s, cc_d)

    @pl.when(cid == 1)
    def _():
        do_rel(y_sc, sc_s, sc_d)

    plsc.subcore_barrier()

    @pl.when(cid == 0)
    def _():
        rd(agg_cc, PER_C, LAST_C)

    @pl.when(cid == 1)
    def _():
        rd(agg_sc, PER_C, LAST_C)


_BLK = 1000


def _norm_col(deg_blk):
    return lax.rsqrt(jnp.maximum(deg_blk[:, :1], 1.0))


def _prep_body(feat_ref, dega_ref, degb_ref, ya_ref, yb_ref):
    f = feat_ref[...]
    ya_ref[...] = f * _norm_col(dega_ref[...])
    yb_ref[...] = f * _norm_col(degb_ref[...])


def _prep(feat, dega, degb):
    n = feat.shape[0]
    bs_x = pl.BlockSpec((_BLK, D), lambda i: (i, 0))
    bs_deg = pl.BlockSpec((_BLK, D), lambda i: (i, 0))
    return pl.pallas_call(
        _prep_body,
        grid=(n // _BLK,),
        in_specs=[bs_x, bs_deg, bs_deg],
        out_specs=[bs_x, bs_x],
        out_shape=[_f32((n, D)), _f32((n, D))],
    )(feat, dega, degb)


def _dense1_body(aa_ref, ab_ref, dia_ref, dib_ref, wa_ref, wb_ref,
                 ba_ref, bb_ref, doa_ref, dob_ref, ya_ref, yb_ref):
    a = aa_ref[...] * _norm_col(dia_ref[...])
    b = ab_ref[...] * _norm_col(dib_ref[...])
    h = (jnp.dot(a, wa_ref[...], preferred_element_type=jnp.float32)
         + jnp.dot(b, wb_ref[...], preferred_element_type=jnp.float32)
         + ba_ref[...] + bb_ref[...])
    h = jnp.maximum(h, 0.0)
    ya_ref[...] = h * _norm_col(doa_ref[...])
    yb_ref[...] = h * _norm_col(dob_ref[...])


def _dense1(agg_a, agg_b, deg_in_a, deg_in_b, wa, wb, ba, bb,
            deg_out_a, deg_out_b):
    n = agg_a.shape[0]
    bs_x = pl.BlockSpec((_BLK, D), lambda i: (i, 0))
    bs_deg = pl.BlockSpec((_BLK, D), lambda i: (i, 0))
    bs_w = pl.BlockSpec((D, D), lambda i: (0, 0))
    bs_b = pl.BlockSpec((1, D), lambda i: (0, 0))
    return pl.pallas_call(
        _dense1_body,
        grid=(n // _BLK,),
        in_specs=[bs_x, bs_x, bs_deg, bs_deg, bs_w, bs_w, bs_b, bs_b,
                  bs_deg, bs_deg],
        out_specs=[bs_x, bs_x],
        out_shape=[_f32((n, D)), _f32((n, D))],
    )(agg_a, agg_b, deg_in_a, deg_in_b, wa, wb, ba, bb,
      deg_out_a, deg_out_b)


def _final_s_body(aa_ref, ab_ref, dia_ref, dib_ref, wa_ref, wb_ref,
                  ba_ref, bb_ref, out_ref):
    i = pl.program_id(0)
    a = aa_ref[...] * _norm_col(dia_ref[...])
    b = ab_ref[...] * _norm_col(dib_ref[...])
    h = (jnp.dot(a, wa_ref[...], preferred_element_type=jnp.float32)
         + jnp.dot(b, wb_ref[...], preferred_element_type=jnp.float32)
         + ba_ref[...] + bb_ref[...])
    m = jnp.max(h, axis=0, keepdims=True)

    @pl.when(i == 0)
    def _():
        out_ref[...] = m

    @pl.when(i > 0)
    def _():
        out_ref[...] = jnp.maximum(out_ref[...], m)


def _final_s(agg_a, agg_b, deg_in_a, deg_in_b, wa, wb, ba, bb):
    bs_x = pl.BlockSpec((_BLK, D), lambda i: (i, 0))
    bs_deg = pl.BlockSpec((_BLK, D), lambda i: (i, 0))
    bs_w = pl.BlockSpec((D, D), lambda i: (0, 0))
    bs_b = pl.BlockSpec((1, D), lambda i: (0, 0))
    return pl.pallas_call(
        _final_s_body,
        grid=(N_S // _BLK,),
        in_specs=[bs_x, bs_x, bs_deg, bs_deg, bs_w, bs_w, bs_b, bs_b],
        out_specs=pl.BlockSpec((1, D), lambda i: (0, 0)),
        out_shape=_f32((1, D)),
    )(agg_a, agg_b, deg_in_a, deg_in_b, wa, wb, ba, bb)


def _final_c_body(aa_ref, ab_ref, dia_ref, dib_ref, wa_ref, wb_ref,
                  ba_ref, bb_ref, out_ref):
    a = aa_ref[...] * _norm_col(dia_ref[...])
    b = ab_ref[...] * _norm_col(dib_ref[...])
    out_ref[...] = (jnp.dot(a, wa_ref[...], preferred_element_type=jnp.float32)
                    + jnp.dot(b, wb_ref[...], preferred_element_type=jnp.float32)
                    + ba_ref[...] + bb_ref[...])


def _final_c(agg_a, agg_b, deg_in_a, deg_in_b, wa, wb, ba, bb):
    bs_x = pl.BlockSpec((_BLK, D), lambda i: (i, 0))
    bs_deg = pl.BlockSpec((_BLK, D), lambda i: (i, 0))
    bs_w = pl.BlockSpec((D, D), lambda i: (0, 0))
    bs_b = pl.BlockSpec((1, D), lambda i: (0, 0))
    return pl.pallas_call(
        _final_c_body,
        grid=(N_C // _BLK,),
        in_specs=[bs_x, bs_x, bs_deg, bs_deg, bs_w, bs_w, bs_b, bs_b],
        out_specs=bs_x,
        out_shape=_f32((N_C, D)),
    )(agg_a, agg_b, deg_in_a, deg_in_b, wa, wb, ba, bb)


def kernel(feat_sentence, feat_context, ss_src, ss_dst, sc_src, sc_dst,
           cs_src, cs_dst, cc_src, cc_dst,
           W1_ss, b1_ss, W1_sc, b1_sc, W1_cs, b1_cs, W1_cc, b1_cc,
           W2_ss, b2_ss, W2_sc, b2_sc, W2_cs, b2_cs, W2_cc, b2_cc):
    def pad2d(idx, fill):
        idx = idx.astype(jnp.int32)
        pad = jnp.full((E_PAD - E,), fill, jnp.int32)
        return jnp.concatenate([idx, pad])


    ss_sd, ss_d2 = pad2d(ss_src, N_S), pad2d(ss_dst, N_S)
    sc_sd, sc_d2 = pad2d(sc_src, N_S), pad2d(sc_dst, N_C)
    cs_sd, cs_d2 = pad2d(cs_src, N_C), pad2d(cs_dst, N_S)
    cc_sd, cc_d2 = pad2d(cc_src, N_C), pad2d(cc_dst, N_C)
    ss_s2, sc_s2 = pad2d(ss_src, 0), pad2d(sc_src, 0)
    cs_s2, cc_s2 = pad2d(cs_src, 0), pad2d(cc_src, 0)

    ones128 = jnp.ones((CHUNK, D), jnp.float32)
    zeros128 = jnp.zeros((ZS, D), jnp.float32)

    d_ss_dst, d_sc_dst, d_cs_dst, d_cc_dst = _hist2(
        ss_d2, sc_d2, cs_d2, cc_d2, ones128, zeros128)
    d_ss_src, d_cs_src, d_sc_src, d_cc_src = _hist2(
        ss_sd, cs_sd, sc_sd, cc_sd, ones128, zeros128)

    y_ss, y_sc = _prep(feat_sentence, d_ss_src, d_sc_src)
    y_cs, y_cc = _prep(feat_context, d_cs_src, d_cc_src)

    agg_ss, agg_sc, agg_cs, agg_cc = _aggregate(
        y_ss, y_sc, y_cs, y_cc,
        ss_s2, ss_d2, sc_s2, sc_d2, cs_s2, cs_d2, cc_s2, cc_d2, zeros128)

    b1_ss_r, b1_sc_r = b1_ss.reshape(1, D), b1_sc.reshape(1, D)
    b1_cs_r, b1_cc_r = b1_cs.reshape(1, D), b1_cc.reshape(1, D)
    b2_ss_r, b2_sc_r = b2_ss.reshape(1, D), b2_sc.reshape(1, D)
    b2_cs_r, b2_cc_r = b2_cs.reshape(1, D), b2_cc.reshape(1, D)

    y2_ss, y2_sc = _dense1(agg_ss, agg_cs, d_ss_dst, d_cs_dst,
                           W1_ss, W1_cs, b1_ss_r, b1_cs_r,
                           d_ss_src, d_sc_src)
    y2_cs, y2_cc = _dense1(agg_sc, agg_cc, d_sc_dst, d_cc_dst,
                           W1_sc, W1_cc, b1_sc_r, b1_cc_r,
                           d_cs_src, d_cc_src)

    agg2_ss, agg2_sc, agg2_cs, agg2_cc = _aggregate(
        y2_ss, y2_sc, y2_cs, y2_cc,
        ss_s2, ss_d2, sc_s2, sc_d2, cs_s2, cs_d2, cc_s2, cc_d2, zeros128)

    doc = _final_s(agg2_ss, agg2_cs, d_ss_dst, d_cs_dst,
                   W2_ss, W2_cs, b2_ss_r, b2_cs_r)
    h_c = _final_c(agg2_sc, agg2_cc, d_sc_dst, d_cc_dst,
                   W2_sc, W2_cc, b2_sc_r, b2_cc_r)
    return (doc, h_c)


# speedup vs baseline: 1.0862x; 1.0286x over previous
import functools

import jax
import jax.numpy as jnp
from jax import lax
from jax.experimental import pallas as pl
from jax.experimental.pallas import tpu as pltpu
from jax.experimental.pallas import tpu_sc as plsc

N_S, N_C, E, D = 8000, 2000, 80000, 128

NC, NS, CHUNK = 2, 16, 256
E_PAD = 81920
NCHW = E_PAD // (NS * CHUNK)
EPW = E_PAD // NS
TS_ROWS, TC_ROWS = 8064, 2048
ZS, ZC = TS_ROWS // NS, TC_ROWS // NS

PER_S, LAST_S = 504, N_S - 15 * 504
PER_C, LAST_C = 128, N_C - 15 * 128


def _mo8(x):
    return pl.multiple_of(x, 8)

_mesh = plsc.VectorSubcoreMesh(core_axis_name="c", subcore_axis_name="s",
                               num_cores=NC, num_subcores=NS)


def _f32(shape):
    return jax.ShapeDtypeStruct(shape, jnp.float32)


LS = 32


@functools.partial(
    pl.kernel,
    out_type=[_f32((N_S, D)), _f32((N_C, D))],
    mesh=_mesh,
    scratch_types=[
        pltpu.VMEM_SHARED((TS_ROWS, D), jnp.float32),
        pltpu.VMEM_SHARED((TC_ROWS, D), jnp.float32),
        pltpu.VMEM((EPW,), jnp.int32),
        pltpu.VMEM((CHUNK, D), jnp.float32),
    ],
)
def _hist(es0, es1, es2, es3, ec0, ec1, ec2, ec3, masks_hbm, zeros_hbm,
          hs, hc, ts_tab, tc_tab, idx_v, ones_v):
    sid = lax.axis_index("s")
    cid = lax.axis_index("c")

    pltpu.sync_copy(zeros_hbm.at[pl.ds(0, ZS)],
                    ts_tab.at[pl.ds(_mo8(sid * ZS), ZS)])
    pltpu.sync_copy(zeros_hbm.at[pl.ds(0, ZC)],
                    tc_tab.at[pl.ds(_mo8(sid * ZC), ZC)])
    plsc.subcore_barrier()

    def hist(e1d, table, g):
        pltpu.sync_copy(masks_hbm.at[pl.ds(g * CHUNK, CHUNK)], ones_v)
        pltpu.sync_copy(e1d.at[pl.ds(pl.multiple_of(sid * EPW, 256), EPW)],
                        idx_v)

        @pl.loop(0, NCHW)
        def _(j):
            sl = pl.ds(pl.multiple_of(j * CHUNK, 256), CHUNK)
            pltpu.sync_copy(ones_v, table.at[idx_v.at[sl]], add=True)

    @pl.when(cid == 0)
    def _():
        hist(es0, ts_tab, 0)
        hist(es1, ts_tab, 1)
        hist(es2, ts_tab, 2)
        hist(es3, ts_tab, 3)

    @pl.when(cid == 1)
    def _():
        hist(ec0, tc_tab, 0)
        hist(ec1, tc_tab, 1)
        hist(ec2, tc_tab, 2)
        hist(ec3, tc_tab, 3)

    plsc.subcore_barrier()

    def rd(table, out, per, last):
        @pl.when(sid < NS - 1)
        def _():
            off = _mo8(sid * per)
            pltpu.sync_copy(table.at[pl.ds(off, per)], out.at[pl.ds(off, per)])

        @pl.when(sid == NS - 1)
        def _():
            off = (NS - 1) * per
            pltpu.sync_copy(table.at[pl.ds(off, last)],
                            out.at[pl.ds(off, last)])

    @pl.when(cid == 0)
    def _():
        rd(ts_tab, hs, PER_S, LAST_S)

    @pl.when(cid == 1)
    def _():
        rd(tc_tab, hc, PER_C, LAST_C)


CH_A = 128
NCH_A = EPW // CH_A


@functools.partial(
    pl.kernel,
    out_type=[_f32((N_S, D)), _f32((N_C, D)),
              _f32((N_S, D)), _f32((N_C, D))],
    mesh=_mesh,
    scratch_types=[
        pltpu.VMEM_SHARED((TS_ROWS, D), jnp.float32),
        pltpu.VMEM_SHARED((TC_ROWS, D), jnp.float32),
        pltpu.VMEM((EPW,), jnp.int32),
        pltpu.VMEM((EPW,), jnp.int32),
        pltpu.VMEM((CH_A, D), jnp.float32),
        pltpu.VMEM((CH_A, D), jnp.float32),
        pltpu.SemaphoreType.DMA,
        pltpu.SemaphoreType.DMA,
    ],
)
def _aggregate(y_ss, y_sc, y_cs, y_cc,
               ss_s, ss_d, sc_s, sc_d, cs_s, cs_d, cc_s, cc_d,
               zeros_hbm,
               agg_ss, agg_sc, agg_cs, agg_cc,
               tab, stage, sidx, didx, rows0, rows1,
               sem0, sem1):
    sid = lax.axis_index("s")
    cid = lax.axis_index("c")

    def stripes(per, last, fn):
        @pl.when(sid < NS - 1)
        def _():
            fn(_mo8(sid * per), per)

        @pl.when(sid == NS - 1)
        def _():
            fn((NS - 1) * per, last)

    def rd(out, per, last):
        def f(off, sz):
            pltpu.sync_copy(tab.at[pl.ds(off, sz)], out.at[pl.ds(off, sz)])
        stripes(per, last, f)

    def load_stage(y_hbm, per, last):
        def f(off, sz):
            pltpu.sync_copy(y_hbm.at[pl.ds(off, sz)],
                            stage.at[pl.ds(off, sz)])
        stripes(per, last, f)

    def do_rel(src, s1d, d1d):
        base = pl.ds(pl.multiple_of(sid * EPW, 256), EPW)
        pltpu.sync_copy(s1d.at[base], sidx)
        pltpu.sync_copy(d1d.at[base], didx)

        def g(j):
            return pl.ds(pl.multiple_of(j * CH_A, CH_A), CH_A)

        pltpu.make_async_copy(src.at[sidx.at[g(0)]], rows0, sem0).start()

        @pl.loop(0, NCH_A, step=2)
        def _(j):
            pltpu.make_async_copy(src.at[sidx.at[g(j)]], rows0, sem0).wait()
            pltpu.make_async_copy(src.at[sidx.at[g(j + 1)]], rows1,
                                  sem1).start()
            pltpu.sync_copy(rows0, tab.at[didx.at[g(j)]], add=True)
            pltpu.make_async_copy(src.at[sidx.at[g(j + 1)]], rows1,
                                  sem1).wait()

            @pl.when(j + 2 < NCH_A)
            def _():
                pltpu.make_async_copy(src.at[sidx.at[g(j + 2)]], rows0,
                                      sem0).start()

            pltpu.sync_copy(rows1, tab.at[didx.at[g(j + 1)]], add=True)

    pltpu.sync_copy(zeros_hbm.at[pl.ds(0, ZS)],
                    tab.at[pl.ds(_mo8(sid * ZS), ZS)])

    @pl.when(cid == 1)
    def _():
        load_stage(y_cs, PER_C, LAST_C)

    plsc.subcore_barrier()

    @pl.when(cid == 0)
    def _():
        do_rel(y_ss, ss_s, ss_d)

    @pl.when(cid == 1)
    def _():
        do_rel(stage, cs_s, cs_d)

    plsc.subcore_barrier()

    @pl.when(cid == 0)
    def _():
        rd(agg_ss, PER_S, LAST_S)
        load_stage(y_cc, PER_C, LAST_C)

    @pl.when(cid == 1)
    def _():
        rd(agg_cs, PER_S, LAST_S)

    plsc.subcore_barrier()

    pltpu.sync_copy(zeros_hbm.at[pl.ds(0, ZC)],
                    tab.at[pl.ds(_mo8(sid * ZC), ZC)])
    plsc.subcore_barrier()

    @pl.when(cid == 0)
    def _():
        do_rel(stage, cc_s, cc_d)

    @pl.when(cid == 1)
    def _():
        do_rel(y_sc, sc_s, sc_d)

    plsc.subcore_barrier()

    @pl.when(cid == 0)
    def _():
        rd(agg_cc, PER_C, LAST_C)

    @pl.when(cid == 1)
    def _():
        rd(agg_sc, PER_C, LAST_C)


_BLK = 1000


def _norm_col(h_blk, c):
    return lax.rsqrt(jnp.maximum(h_blk[:, c:c + 1], 1.0))


def _prep(feat, h, ca, cb):
    def body(feat_ref, h_ref, ya_ref, yb_ref):
        f = feat_ref[...]
        hv = h_ref[...]
        ya_ref[...] = f * _norm_col(hv, ca)
        yb_ref[...] = f * _norm_col(hv, cb)

    n = feat.shape[0]
    bs_x = pl.BlockSpec((_BLK, D), lambda i: (i, 0))
    return pl.pallas_call(
        body,
        grid=(n // _BLK,),
        in_specs=[bs_x, bs_x],
        out_specs=[bs_x, bs_x],
        out_shape=[_f32((n, D)), _f32((n, D))],
    )(feat, h)


def _dense1(agg_a, agg_b, h, wa, wb, ba, bb, cia, cib, coa, cob):
    def body(aa_ref, ab_ref, h_ref, wa_ref, wb_ref, ba_ref, bb_ref,
             ya_ref, yb_ref):
        hv = h_ref[...]
        a = aa_ref[...] * _norm_col(hv, cia)
        b = ab_ref[...] * _norm_col(hv, cib)
        y = (jnp.dot(a, wa_ref[...], preferred_element_type=jnp.float32)
             + jnp.dot(b, wb_ref[...], preferred_element_type=jnp.float32)
             + ba_ref[...] + bb_ref[...])
        y = jnp.maximum(y, 0.0)
        ya_ref[...] = y * _norm_col(hv, coa)
        yb_ref[...] = y * _norm_col(hv, cob)

    n = agg_a.shape[0]
    bs_x = pl.BlockSpec((_BLK, D), lambda i: (i, 0))
    bs_w = pl.BlockSpec((D, D), lambda i: (0, 0))
    bs_b = pl.BlockSpec((1, D), lambda i: (0, 0))
    return pl.pallas_call(
        body,
        grid=(n // _BLK,),
        in_specs=[bs_x, bs_x, bs_x, bs_w, bs_w, bs_b, bs_b],
        out_specs=[bs_x, bs_x],
        out_shape=[_f32((n, D)), _f32((n, D))],
    )(agg_a, agg_b, h, wa, wb, ba, bb)


def _final_s(agg_a, agg_b, h, wa, wb, ba, bb, cia, cib):
    def body(aa_ref, ab_ref, h_ref, wa_ref, wb_ref, ba_ref, bb_ref, out_ref):
        i = pl.program_id(0)
        hv = h_ref[...]
        a = aa_ref[...] * _norm_col(hv, cia)
        b = ab_ref[...] * _norm_col(hv, cib)
        y = (jnp.dot(a, wa_ref[...], preferred_element_type=jnp.float32)
             + jnp.dot(b, wb_ref[...], preferred_element_type=jnp.float32)
             + ba_ref[...] + bb_ref[...])
        m = jnp.max(y, axis=0, keepdims=True)

        @pl.when(i == 0)
        def _():
            out_ref[...] = m

        @pl.when(i > 0)
        def _():
            out_ref[...] = jnp.maximum(out_ref[...], m)

    bs_x = pl.BlockSpec((_BLK, D), lambda i: (i, 0))
    bs_w = pl.BlockSpec((D, D), lambda i: (0, 0))
    bs_b = pl.BlockSpec((1, D), lambda i: (0, 0))
    return pl.pallas_call(
        body,
        grid=(N_S // _BLK,),
        in_specs=[bs_x, bs_x, bs_x, bs_w, bs_w, bs_b, bs_b],
        out_specs=pl.BlockSpec((1, D), lambda i: (0, 0)),
        out_shape=_f32((1, D)),
    )(agg_a, agg_b, h, wa, wb, ba, bb)


def _final_c(agg_a, agg_b, h, wa, wb, ba, bb, cia, cib):
    def body(aa_ref, ab_ref, h_ref, wa_ref, wb_ref, ba_ref, bb_ref, out_ref):
        hv = h_ref[...]
        a = aa_ref[...] * _norm_col(hv, cia)
        b = ab_ref[...] * _norm_col(hv, cib)
        out_ref[...] = (
            jnp.dot(a, wa_ref[...], preferred_element_type=jnp.float32)
            + jnp.dot(b, wb_ref[...], preferred_element_type=jnp.float32)
            + ba_ref[...] + bb_ref[...])

    bs_x = pl.BlockSpec((_BLK, D), lambda i: (i, 0))
    bs_w = pl.BlockSpec((D, D), lambda i: (0, 0))
    bs_b = pl.BlockSpec((1, D), lambda i: (0, 0))
    return pl.pallas_call(
        body,
        grid=(N_C // _BLK,),
        in_specs=[bs_x, bs_x, bs_x, bs_w, bs_w, bs_b, bs_b],
        out_specs=bs_x,
        out_shape=_f32((N_C, D)),
    )(agg_a, agg_b, h, wa, wb, ba, bb)


def kernel(feat_sentence, feat_context, ss_src, ss_dst, sc_src, sc_dst,
           cs_src, cs_dst, cc_src, cc_dst,
           W1_ss, b1_ss, W1_sc, b1_sc, W1_cs, b1_cs, W1_cc, b1_cc,
           W2_ss, b2_ss, W2_sc, b2_sc, W2_cs, b2_cs, W2_cc, b2_cc):
    def pad2d(idx, fill):
        idx = idx.astype(jnp.int32)
        pad = jnp.full((E_PAD - E,), fill, jnp.int32)
        return jnp.concatenate([idx, pad])


    ss_sd, ss_d2 = pad2d(ss_src, N_S), pad2d(ss_dst, N_S)
    sc_sd, sc_d2 = pad2d(sc_src, N_S), pad2d(sc_dst, N_C)
    cs_sd, cs_d2 = pad2d(cs_src, N_C), pad2d(cs_dst, N_S)
    cc_sd, cc_d2 = pad2d(cc_src, N_C), pad2d(cc_dst, N_C)
    ss_s2, sc_s2 = pad2d(ss_src, 0), pad2d(sc_src, 0)
    cs_s2, cc_s2 = pad2d(cs_src, 0), pad2d(cc_src, 0)

    lane = jnp.arange(D)[None, :]
    grp = jnp.arange(4)[:, None]
    masks = ((lane >= grp * LS) & (lane < (grp + 1) * LS)).astype(jnp.float32)
    masks_hbm = jnp.repeat(masks, CHUNK, axis=0)
    zeros128 = jnp.zeros((ZS, D), jnp.float32)

    hs, hc = _hist(ss_d2, ss_sd, sc_sd, cs_d2,
                   sc_d2, cs_sd, cc_sd, cc_d2, masks_hbm, zeros128)

    y_ss, y_sc = _prep(feat_sentence, hs, 1 * LS, 2 * LS)
    y_cs, y_cc = _prep(feat_context, hc, 1 * LS, 2 * LS)

    agg_ss, agg_sc, agg_cs, agg_cc = _aggregate(
        y_ss, y_sc, y_cs, y_cc,
        ss_s2, ss_d2, sc_s2, sc_d2, cs_s2, cs_d2, cc_s2, cc_d2, zeros128)

    b1_ss_r, b1_sc_r = b1_ss.reshape(1, D), b1_sc.reshape(1, D)
    b1_cs_r, b1_cc_r = b1_cs.reshape(1, D), b1_cc.reshape(1, D)
    b2_ss_r, b2_sc_r = b2_ss.reshape(1, D), b2_sc.reshape(1, D)
    b2_cs_r, b2_cc_r = b2_cs.reshape(1, D), b2_cc.reshape(1, D)

    y2_ss, y2_sc = _dense1(agg_ss, agg_cs, hs,
                           W1_ss, W1_cs, b1_ss_r, b1_cs_r,
                           0 * LS, 3 * LS, 1 * LS, 2 * LS)
    y2_cs, y2_cc = _dense1(agg_sc, agg_cc, hc,
                           W1_sc, W1_cc, b1_sc_r, b1_cc_r,
                           0 * LS, 3 * LS, 1 * LS, 2 * LS)

    agg2_ss, agg2_sc, agg2_cs, agg2_cc = _aggregate(
        y2_ss, y2_sc, y2_cs, y2_cc,
        ss_s2, ss_d2, sc_s2, sc_d2, cs_s2, cs_d2, cc_s2, cc_d2, zeros128)

    doc = _final_s(agg2_ss, agg2_cs, hs,
                   W2_ss, W2_cs, b2_ss_r, b2_cs_r, 0 * LS, 3 * LS)
    h_c = _final_c(agg2_sc, agg2_cc, hc,
                   W2_sc, W2_cc, b2_sc_r, b2_cc_r, 0 * LS, 3 * LS)
    return (doc, h_c)
